# Initial kernel scaffold; baseline (speedup 1.0000x reference)
import functools

import jax
import jax.numpy as jnp
from jax import lax
from jax.experimental import pallas as pl
from jax.experimental.pallas import tpu as pltpu
from jax.experimental.pallas import tpu_sc as plsc

NC = 2
NS = 16
L = 16
NW = NC * NS

B = 16384
K = 10
D = 64

BPW = B // NW
CHUNK = 64
NCHUNK = BPW // CHUNK
NGRP = CHUNK // L
NSU = CHUNK * K // 128


def _sc_body(cen_hbm, ctx_hbm, ns_hbm, wc_hbm, wx_hbm, out_hbm,
             cen_idx, ctx_idx, ns_idx, score_v,
             c_rows0, c_rows1, x_rows0, x_rows1, n_rows0, n_rows1,
             sem0, sem1):
  wid = lax.axis_index("s") * NC + lax.axis_index("c")

  pltpu.sync_copy(cen_hbm.at[wid], cen_idx)
  pltpu.sync_copy(ctx_hbm.at[wid], ctx_idx)
  pltpu.sync_copy(ns_hbm.at[wid], ns_idx)

  bufs = ((c_rows0, x_rows0, n_rows0, sem0),
          (c_rows1, x_rows1, n_rows1, sem1))

  def issue(g):
    c_b, x_b, n_b, sem = bufs[g % 2]
    cps = [
        pltpu.async_copy(wc_hbm.at[cen_idx.at[g]], c_b, sem),
        pltpu.async_copy(wx_hbm.at[ctx_idx.at[g]], x_b, sem),
    ]
    for u in range(NSU):
      cps.append(pltpu.async_copy(
          wx_hbm.at[ns_idx.at[g * NSU + u]],
          n_b.at[pl.ds(u * 128, 128)], sem))
    return cps

  iota = lax.iota(jnp.int32, L)
  pending = issue(0)

  for g in range(NCHUNK):
    nxt = issue(g + 1) if g + 1 < NCHUNK else None
    for cp in pending:
      cp.wait()
    pending = nxt

    c_b, x_b, n_b, _ = bufs[g % 2]
    for grp in range(NGRP):
      row = grp * L + iota
      nrow = [row * K + k for k in range(K)]

      def body(d, accs):
        dv = jnp.full((L,), d, jnp.int32)
        cv = plsc.load_gather(c_b, [row, dv])
        xv = plsc.load_gather(x_b, [row, dv])
        new = [accs[0] + cv * xv]
        for k in range(K):
          nv = plsc.load_gather(n_b, [nrow[k], dv])
          new.append(accs[k + 1] + cv * nv)
        return tuple(new)

      accs = lax.fori_loop(
          0, D, body, tuple(jnp.zeros((L,), jnp.float32) for _ in range(K + 1)))

      off = g * CHUNK + grp * L
      score_v[0, pl.ds(off, L)] = -accs[0]
      for k in range(K):
        score_v[1 + k, pl.ds(off, L)] = accs[k + 1]

  pltpu.sync_copy(score_v, out_hbm.at[wid])


def _tc_body(s_ref, o_ref):
  x = s_ref[...]
  o_ref[0, 0] = jnp.sum(jnp.maximum(x, 0.0) +
                        jnp.log1p(jnp.exp(-jnp.abs(x))))


@jax.jit
def kernel(center, context, ns, W_center, W_context):
  cen = center.astype(jnp.int32).reshape(NW, NCHUNK, CHUNK)
  ctx = context.astype(jnp.int32).reshape(NW, NCHUNK, CHUNK)
  nsr = ns.astype(jnp.int32).reshape(NW, NCHUNK * NSU, 128)

  mesh = plsc.VectorSubcoreMesh(core_axis_name="c", subcore_axis_name="s")
  scores = pl.kernel(
      _sc_body,
      out_type=jax.ShapeDtypeStruct((NW, 1 + K, BPW), jnp.float32),
      mesh=mesh,
      scratch_types=[
          pltpu.VMEM((NCHUNK, CHUNK), jnp.int32),
          pltpu.VMEM((NCHUNK, CHUNK), jnp.int32),
          pltpu.VMEM((NCHUNK * NSU, 128), jnp.int32),
          pltpu.VMEM((1 + K, BPW), jnp.float32),
          pltpu.VMEM((CHUNK, D), jnp.float32),
          pltpu.VMEM((CHUNK, D), jnp.float32),
          pltpu.VMEM((CHUNK, D), jnp.float32),
          pltpu.VMEM((CHUNK, D), jnp.float32),
          pltpu.VMEM((CHUNK * K, D), jnp.float32),
          pltpu.VMEM((CHUNK * K, D), jnp.float32),
          pltpu.SemaphoreType.DMA,
          pltpu.SemaphoreType.DMA,
      ],
  )(cen, ctx, nsr, W_center, W_context)

  loss = pl.pallas_call(
      _tc_body,
      out_shape=jax.ShapeDtypeStruct((1, 1), jnp.float32),
      out_specs=pl.BlockSpec(memory_space=pltpu.SMEM),
  )(scores.reshape(NW * (1 + K), BPW))
  return loss[0, 0]


# speedup vs baseline: 2.5754x; 2.5754x over previous
import functools

import jax
import jax.numpy as jnp
from jax import lax
from jax.experimental import pallas as pl
from jax.experimental.pallas import tpu as pltpu
from jax.experimental.pallas import tpu_sc as plsc

NC = 2
NS = 16
L = 16
NW = NC * NS

B = 16384
K = 10
D = 64

BPW = B // NW
CHUNK = 64
NCHUNK = BPW // CHUNK
NGRP = CHUNK // L
NSU = CHUNK * K // 128


def _sc_body(cen_hbm, ctx_hbm, ns_hbm, wc_hbm, wx_hbm, out_hbm,
             cen_idx, ctx_idx, ns_idx, score_v,
             c_rows0, c_rows1, x_rows0, x_rows1, n_rows0, n_rows1,
             sem0, sem1):
  wid = lax.axis_index("s") * NC + lax.axis_index("c")

  pltpu.sync_copy(cen_hbm.at[wid], cen_idx)
  pltpu.sync_copy(ctx_hbm.at[wid], ctx_idx)
  pltpu.sync_copy(ns_hbm.at[wid], ns_idx)

  bufs = ((c_rows0, x_rows0, n_rows0, sem0),
          (c_rows1, x_rows1, n_rows1, sem1))

  def issue(g):
    c_b, x_b, n_b, sem = bufs[g % 2]
    cps = [
        pltpu.async_copy(wc_hbm.at[cen_idx.at[g]], c_b, sem),
        pltpu.async_copy(wx_hbm.at[ctx_idx.at[g]], x_b, sem),
    ]
    for u in range(NSU):
      cps.append(pltpu.async_copy(
          wx_hbm.at[ns_idx.at[g * NSU + u]],
          n_b.at[pl.ds(u * 128, 128)], sem))
    return cps

  iota = lax.iota(jnp.int32, L)
  pending = issue(0)

  for g in range(NCHUNK):
    nxt = issue(g + 1) if g + 1 < NCHUNK else None
    for cp in pending:
      cp.wait()
    pending = nxt

    c_b, x_b, n_b, _ = bufs[g % 2]
    for grp in range(NGRP):
      row = grp * L + iota
      nrow = [row * K + k for k in range(K)]

      def body(d, accs):
        dv = jnp.full((L,), d, jnp.int32)
        cv = plsc.load_gather(c_b, [row, dv])
        xv = plsc.load_gather(x_b, [row, dv])
        new = [accs[0] + cv * xv]
        for k in range(K):
          nv = plsc.load_gather(n_b, [nrow[k], dv])
          new.append(accs[k + 1] + cv * nv)
        return tuple(new)

      accs = lax.fori_loop(
          0, D, body, tuple(jnp.zeros((L,), jnp.float32) for _ in range(K + 1)))

      off = g * CHUNK + grp * L
      score_v[0, pl.ds(off, L)] = -accs[0]
      for k in range(K):
        score_v[1 + k, pl.ds(off, L)] = accs[k + 1]

  pltpu.sync_copy(score_v, out_hbm.at[wid])


def _tc_body(s_ref, o_ref):
  x = s_ref[...]
  o_ref[0, 0] = jnp.sum(jnp.maximum(x, 0.0) +
                        jnp.log1p(jnp.exp(-jnp.abs(x))))


@jax.jit
def kernel(center, context, ns, W_center, W_context):
  cen = center.astype(jnp.int32).reshape(NW, NCHUNK, CHUNK)
  ctx = context.astype(jnp.int32).reshape(NW, NCHUNK, CHUNK)
  nsr = ns.astype(jnp.int32).reshape(NW, NCHUNK * NSU, 128)

  mesh = plsc.VectorSubcoreMesh(core_axis_name="c", subcore_axis_name="s")
  scores = pl.kernel(
      _sc_body,
      out_type=jax.ShapeDtypeStruct((NW, 1 + K, BPW), jnp.float32),
      mesh=mesh,
      compiler_params=pltpu.CompilerParams(
          needs_layout_passes=False, use_tc_tiling_on_sc=False),
      scratch_types=[
          pltpu.VMEM((NCHUNK, CHUNK), jnp.int32),
          pltpu.VMEM((NCHUNK, CHUNK), jnp.int32),
          pltpu.VMEM((NCHUNK * NSU, 128), jnp.int32),
          pltpu.VMEM((1 + K, BPW), jnp.float32),
          pltpu.VMEM((CHUNK, D), jnp.float32),
          pltpu.VMEM((CHUNK, D), jnp.float32),
          pltpu.VMEM((CHUNK, D), jnp.float32),
          pltpu.VMEM((CHUNK, D), jnp.float32),
          pltpu.VMEM((CHUNK * K, D), jnp.float32),
          pltpu.VMEM((CHUNK * K, D), jnp.float32),
          pltpu.SemaphoreType.DMA,
          pltpu.SemaphoreType.DMA,
      ],
  )(cen, ctx, nsr, W_center, W_context)

  loss = pl.pallas_call(
      _tc_body,
      out_shape=jax.ShapeDtypeStruct((1, 1), jnp.float32),
      out_specs=pl.BlockSpec(memory_space=pltpu.SMEM),
  )(scores.reshape(NW * (1 + K), BPW))
  return loss[0, 0]
